# baseline (speedup 1.0000x reference)
import functools

import jax
import jax.numpy as jnp
from jax import lax
from jax.experimental import pallas as pl
from jax.experimental.pallas import tpu as pltpu
from jax.experimental.pallas import tpu_sc as plsc

HIDDEN = 256
N_NODES = 100000

NC = 2
NS = 16
NW = NC * NS
LANES = 16
HBLK = HIDDEN // LANES

GPC = 7
CHUNK = GPC * LANES
NCHUNK = 28
B_PER_W = CHUNK * NCHUNK
NGROUPS = B_PER_W // LANES
N_PAD = NW * B_PER_W
NBUF = 2

_mesh = plsc.VectorSubcoreMesh(core_axis_name="c", subcore_axis_name="s")


@functools.partial(
    pl.kernel,
    out_type=jax.ShapeDtypeStruct((N_PAD, HIDDEN), jnp.float32),
    mesh=_mesh,
    scratch_types=[
        pltpu.VMEM((NGROUPS, LANES), jnp.int32),
        pltpu.VMEM((2, HIDDEN), jnp.float32),
        pltpu.VMEM((NBUF, CHUNK, HIDDEN), jnp.float32),
        pltpu.VMEM_SHARED((NS, CHUNK, HIDDEN), jnp.float32),
        pltpu.SemaphoreType.DMA,
        pltpu.SemaphoreType.DMA,
    ],
)
def _sc_lookup(w_hbm, idx_hbm, out_hbm, idx_v, w_v, rows_v, spb, sem_d, sem_s):
    wid = lax.axis_index("s") * NC + lax.axis_index("c")
    sid = lax.axis_index("s")
    base = wid * B_PER_W
    pltpu.sync_copy(idx_hbm.at[wid], idx_v)
    pltpu.sync_copy(w_hbm, w_v)

    w0 = [w_v[0, pl.ds(h * LANES, LANES)] for h in range(HBLK)]
    w1 = [w_v[1, pl.ds(h * LANES, LANES)] for h in range(HBLK)]
    my_spb = spb.at[sid]

    def out_slice(j):
        return out_hbm.at[pl.ds(base + j * CHUNK, CHUNK)]

    def compute_chunk(j, buf):
        rows = rows_v.at[buf]

        def group(g2, carry):
            v = idx_v[j * GPC + g2]
            for r in range(LANES):
                pred = v[r] == 0
                for h in range(HBLK):
                    rows[g2 * LANES + r, pl.ds(h * LANES, LANES)] = (
                        jnp.where(pred, w0[h], w1[h])
                    )
            return carry

        lax.fori_loop(0, GPC, group, 0)

    def body(j, carry):
        buf = lax.rem(j, NBUF)
        even = lax.rem(j, 2) == 0

        @pl.when(jnp.logical_and(even, j >= 2))
        def _():
            pltpu.make_async_copy(
                rows_v.at[buf], out_slice(j - 2), sem_d
            ).wait()

        compute_chunk(j, buf)

        @pl.when(even)
        def _():
            pltpu.async_copy(rows_v.at[buf], out_slice(j), sem_d)

        @pl.when(jnp.logical_not(even))
        def _():
            @pl.when(j >= 3)
            def _():
                pltpu.make_async_copy(my_spb, out_slice(j - 2), sem_s).wait()

            pltpu.sync_copy(rows_v.at[buf], my_spb)
            pltpu.async_copy(my_spb, out_slice(j), sem_s)

        return carry

    lax.fori_loop(0, NCHUNK, body, 0)

    pltpu.make_async_copy(
        rows_v.at[(NCHUNK - 2) % NBUF], out_slice(NCHUNK - 2), sem_d
    ).wait()
    pltpu.make_async_copy(my_spb, out_slice(NCHUNK - 1), sem_s).wait()


def kernel(x, W):
    x = x.astype(jnp.int32)
    x_pad = jnp.zeros((N_PAD,), jnp.int32).at[:N_NODES].set(x)
    x_pad = x_pad.reshape(NW, NGROUPS, LANES)
    out = _sc_lookup(W, x_pad)
    return out[:N_NODES]


# speedup vs baseline: 1.1362x; 1.1362x over previous
import functools

import jax
import jax.numpy as jnp
from jax import lax
from jax.experimental import pallas as pl
from jax.experimental.pallas import tpu as pltpu
from jax.experimental.pallas import tpu_sc as plsc

HIDDEN = 256
N_NODES = 100000

NC = 2
NS = 16
NW = NC * NS
LANES = 16
HBLK = HIDDEN // LANES

GPC = 4
CHUNK = GPC * LANES
NCHUNK = 49
B_PER_W = CHUNK * NCHUNK
NGROUPS = B_PER_W // LANES
N_PAD = NW * B_PER_W
NBUF = 3

_mesh = plsc.VectorSubcoreMesh(core_axis_name="c", subcore_axis_name="s")


@functools.partial(
    pl.kernel,
    out_type=jax.ShapeDtypeStruct((N_PAD, HIDDEN), jnp.float32),
    mesh=_mesh,
    scratch_types=[
        pltpu.VMEM((NGROUPS, LANES), jnp.int32),
        pltpu.VMEM((2, HIDDEN), jnp.float32),
        pltpu.VMEM((NBUF, CHUNK, HIDDEN), jnp.float32),
        pltpu.SemaphoreType.DMA,
    ],
)
def _sc_lookup(w_hbm, idx_hbm, out_hbm, idx_v, w_v, rows_v, sem):
    wid = lax.axis_index("s") * NC + lax.axis_index("c")
    base = wid * B_PER_W
    pltpu.sync_copy(idx_hbm.at[wid], idx_v)
    pltpu.sync_copy(w_hbm, w_v)

    w0 = [w_v[0, pl.ds(h * LANES, LANES)] for h in range(HBLK)]
    w1 = [w_v[1, pl.ds(h * LANES, LANES)] for h in range(HBLK)]

    def out_slice(j):
        return out_hbm.at[pl.ds(base + j * CHUNK, CHUNK)]

    def compute_chunk(j, buf):
        rows = rows_v.at[buf]

        def group(g2, carry):
            v = idx_v[j * GPC + g2]
            for r in range(LANES):
                pred = v[r] == 0
                for h in range(HBLK):
                    rows[g2 * LANES + r, pl.ds(h * LANES, LANES)] = (
                        jnp.where(pred, w0[h], w1[h])
                    )
            return carry

        lax.fori_loop(0, GPC, group, 0)

    def body(j, carry):
        buf = lax.rem(j, NBUF)

        @pl.when(j >= NBUF)
        def _():
            pltpu.make_async_copy(rows_v.at[buf], out_slice(j - NBUF), sem).wait()

        compute_chunk(j, buf)
        pltpu.async_copy(rows_v.at[buf], out_slice(j), sem)
        return carry

    lax.fori_loop(0, NCHUNK, body, 0)

    for k in range(NBUF):
        j = NCHUNK - NBUF + k
        pltpu.make_async_copy(
            rows_v.at[lax.rem(j, NBUF)], out_slice(j), sem
        ).wait()


def kernel(x, W):
    x = x.astype(jnp.int32)
    x_pad = jnp.zeros((N_PAD,), jnp.int32).at[:N_NODES].set(x)
    x_pad = x_pad.reshape(NW, NGROUPS, LANES)
    out = _sc_lookup(W, x_pad)
    return out[:N_NODES]


# speedup vs baseline: 2.3635x; 2.0801x over previous
import functools

import jax
import jax.numpy as jnp
from jax import lax
from jax.experimental import pallas as pl
from jax.experimental.pallas import tpu as pltpu
from jax.experimental.pallas import tpu_sc as plsc

HIDDEN = 256
N_NODES = 100000

NC = 2
NS = 16
NW = NC * NS
LANES = 16
HBLK = HIDDEN // LANES

GPC = 2
CHUNK = GPC * LANES
NCHUNK = 98
B_PER_W = CHUNK * NCHUNK
NGROUPS = B_PER_W // LANES
N_PAD = NW * B_PER_W
LAST_W_ROWS = N_NODES - (NW - 1) * B_PER_W
NCHUNK_LAST = LAST_W_ROWS // CHUNK
NBUF = 3

_mesh = plsc.VectorSubcoreMesh(core_axis_name="c", subcore_axis_name="s")


@functools.partial(
    pl.kernel,
    out_type=jax.ShapeDtypeStruct((N_NODES, HIDDEN), jnp.float32),
    mesh=_mesh,
    scratch_types=[
        pltpu.VMEM((NGROUPS, LANES), jnp.int32),
        pltpu.VMEM((2, HIDDEN), jnp.float32),
        pltpu.VMEM((NBUF, CHUNK, HIDDEN), jnp.float32),
        pltpu.SemaphoreType.DMA,
    ],
)
def _sc_lookup(w_hbm, idx_hbm, out_hbm, idx_v, w_v, rows_v, sem):
    wid = lax.axis_index("s") * NC + lax.axis_index("c")
    base = wid * B_PER_W
    pltpu.sync_copy(idx_hbm.at[wid], idx_v)
    pltpu.sync_copy(w_hbm, w_v)

    w0 = [w_v[0, pl.ds(h * LANES, LANES)] for h in range(HBLK)]
    w1 = [w_v[1, pl.ds(h * LANES, LANES)] for h in range(HBLK)]

    def out_slice(j):
        return out_hbm.at[pl.ds(base + j * CHUNK, CHUNK)]

    def compute_chunk(j, buf):
        rows = rows_v.at[buf]

        def group(g2, carry):
            v = idx_v[j * GPC + g2]
            for r in range(LANES):
                pred = v[r] == 0
                for h in range(HBLK):
                    rows[g2 * LANES + r, pl.ds(h * LANES, LANES)] = (
                        jnp.where(pred, w0[h], w1[h])
                    )
            return carry

        lax.fori_loop(0, GPC, group, 0)

    def body(j, carry):
        buf = lax.rem(j, NBUF)

        @pl.when(j >= NBUF)
        def _():
            pltpu.make_async_copy(rows_v.at[buf], out_slice(j - NBUF), sem).wait()

        compute_chunk(j, buf)
        pltpu.async_copy(rows_v.at[buf], out_slice(j), sem)
        return carry

    nchunk = jnp.where(wid == NW - 1, NCHUNK_LAST, NCHUNK)
    lax.fori_loop(0, nchunk, body, 0)

    def drain(k, carry):
        j = nchunk - NBUF + k
        pltpu.make_async_copy(
            rows_v.at[lax.rem(j, NBUF)], out_slice(j), sem
        ).wait()
        return carry

    lax.fori_loop(0, NBUF, drain, 0)


def kernel(x, W):
    x = x.astype(jnp.int32)
    x_pad = jnp.zeros((N_PAD,), jnp.int32).at[:N_NODES].set(x)
    x_pad = x_pad.reshape(NW, NGROUPS, LANES)
    return _sc_lookup(W, x_pad)
